# baseline (speedup 1.0000x reference)
import jax
import jax.numpy as jnp
from jax import lax
from jax.experimental import pallas as pl
from jax.experimental.pallas import tpu as pltpu
from jax.experimental.pallas import tpu_sc as plsc

N = 10000
E = 320000
D = 128
DE = 16
NW = 32
CH = 80
EPW = E // NW
NSTEP = EPW // CH
NP = 10240
NPT = NP // 16
EQ = E * 16 // 128
CHQ = CH * 16 // 128

import dataclasses as _dc

_MESH = plsc.VectorSubcoreMesh(core_axis_name="c", subcore_axis_name="s")
_SC_PARAMS = pltpu.CompilerParams(use_tc_tiling_on_sc=False)
if "needs_layout_passes" in pltpu.CompilerParams.__dataclass_fields__:
    _SC_PARAMS = _dc.replace(_SC_PARAMS, needs_layout_passes=False)
_F32 = jnp.float32


def _tables_body(h_ref, w1a_ref, w1b_ref, g1_ref, g2_ref):
    h = h_ref[...]
    g1_ref[...] = jnp.dot(h, w1a_ref[...],
                          preferred_element_type=_F32).astype(jnp.bfloat16)
    g2_ref[...] = jnp.dot(h, w1b_ref[...],
                          preferred_element_type=_F32).astype(jnp.bfloat16)


def _build_tables(h, w1a, w1b):
    blk = 2000
    return pl.pallas_call(
        _tables_body,
        grid=(N // blk,),
        in_specs=[
            pl.BlockSpec((blk, D), lambda i: (i, 0)),
            pl.BlockSpec((D, D), lambda i: (0, 0)),
            pl.BlockSpec((D, D), lambda i: (0, 0)),
        ],
        out_specs=[
            pl.BlockSpec((blk, D), lambda i: (i, 0)),
            pl.BlockSpec((blk, D), lambda i: (i, 0)),
        ],
        out_shape=[jax.ShapeDtypeStruct((N, D), jnp.bfloat16)] * 2,
    )(h, w1a, w1b)


def _gather_body(g1_hbm, g2_hbm, p_hbm, row_hbm, col_hbm,
                 s_hbm, dq_hbm,
                 idxr, idxc, bufs_a, bufs_b, bufs_c, bufs_d, bufs_q, bufs_s,
                 gsems, wsems):
    c = lax.axis_index("c")
    s = lax.axis_index("s")
    w = s * 2 + c
    pltpu.sync_copy(row_hbm.at[pl.ds(w * EPW, EPW)], idxr)
    pltpu.sync_copy(col_hbm.at[pl.ds(w * EPW, EPW)], idxc)

    def issue_gather(t, j):
        ir = idxr.at[pl.ds(j * CH, CH)]
        ic = idxc.at[pl.ds(j * CH, CH)]
        pltpu.async_copy(g1_hbm.at[ir], bufs_a[t], gsems.at[t])
        pltpu.async_copy(g2_hbm.at[ic], bufs_b[t], gsems.at[t])
        pltpu.async_copy(p_hbm.at[ir], bufs_c[t], gsems.at[t])
        pltpu.async_copy(p_hbm.at[ic], bufs_d[t], gsems.at[t])

    def wait_gather(t, j):
        ir = idxr.at[pl.ds(j * CH, CH)]
        ic = idxc.at[pl.ds(j * CH, CH)]
        pltpu.make_async_copy(g1_hbm.at[ir], bufs_a[t], gsems.at[t]).wait()
        pltpu.make_async_copy(g2_hbm.at[ic], bufs_b[t], gsems.at[t]).wait()
        pltpu.make_async_copy(p_hbm.at[ir], bufs_c[t], gsems.at[t]).wait()
        pltpu.make_async_copy(p_hbm.at[ic], bufs_d[t], gsems.at[t]).wait()

    def combine(t):
        a, b, pc, pd, q = bufs_a[t], bufs_b[t], bufs_c[t], bufs_d[t], bufs_q[t]
        so = bufs_s[t]

        @pl.loop(0, CHQ)
        def _rows(m):
            for k in range(8):
                i = m * 8 + k
                for cc in range(4):
                    sl = pl.ds(cc * 32, 32)
                    a0, a1 = plsc.unpack(a[i, sl],
                                         format=plsc.PackFormat.INTERLEAVED)
                    b0, b1 = plsc.unpack(b[i, sl],
                                         format=plsc.PackFormat.INTERLEAVED)
                    so[i, pl.ds(cc * 32, 16)] = a0 + b0
                    so[i, pl.ds(cc * 32 + 16, 16)] = a1 + b1
                q[i, :] = pc[i, :] - pd[i, :]

    def issue_writes(t, base):
        pltpu.async_copy(bufs_s[t], s_hbm.at[pl.ds(base, CH)], wsems.at[t])
        pltpu.async_copy(bufs_q[t], dq_hbm.at[pl.ds(base, CH)], wsems.at[t])

    def wait_writes(t, base):
        pltpu.make_async_copy(bufs_s[t], s_hbm.at[pl.ds(base, CH)], wsems.at[t]).wait()
        pltpu.make_async_copy(bufs_q[t], dq_hbm.at[pl.ds(base, CH)], wsems.at[t]).wait()

    issue_gather(0, 0)
    issue_gather(1, 1)

    @pl.loop(0, (NSTEP - 1) // 2)
    def _pair(i):
        for t in range(2):
            jj = 2 * i + t
            base = w * EPW + jj * CH
            wait_gather(t, jj)
            combine(t)
            issue_writes(t, base)
            wait_writes(t, base)

            @pl.when(jj + 2 < NSTEP)
            def _():
                issue_gather(t, jj + 2)

    jj = NSTEP - 1
    base = w * EPW + jj * CH
    wait_gather(0, jj)
    combine(0)
    issue_writes(0, base)
    wait_writes(0, base)


def _gather_sc(g1, g2, pos16, row1d, col1d):
    f = pl.kernel(
        _gather_body,
        out_type=(
            jax.ShapeDtypeStruct((E, D), _F32),
            jax.ShapeDtypeStruct((E, 16), _F32),
        ),
        mesh=_MESH,
        scratch_types=[
            pltpu.VMEM((EPW,), jnp.int32),
            pltpu.VMEM((EPW,), jnp.int32),
            [pltpu.VMEM((CH, D), jnp.bfloat16)] * 2,
            [pltpu.VMEM((CH, D), jnp.bfloat16)] * 2,
            [pltpu.VMEM((CH, 16), _F32)] * 2,
            [pltpu.VMEM((CH, 16), _F32)] * 2,
            [pltpu.VMEM((CH, 16), _F32)] * 2,
            [pltpu.VMEM((CH, D), _F32)] * 2,
            pltpu.SemaphoreType.DMA((2,)),
            pltpu.SemaphoreType.DMA((2,)),
        ],
        compiler_params=_SC_PARAMS,
    )
    return f(g1, g2, pos16, row1d, col1d)


def _edge_body(s_ref, dq_ref, ea_ref,
               w1e_ref, wdsq_ref, b1_ref, w2_ref, b2_ref,
               watt_ref, batt_ref, wp2_ref,
               ef_ref, tr_ref):
    dvec = dq_ref[...]
    dsq = jnp.sum(dvec * dvec, axis=-1, keepdims=True)
    pre = (s_ref[...] + dsq * wdsq_ref[...] + b1_ref[...]
           + jnp.dot(ea_ref[...], w1e_ref[...], preferred_element_type=_F32))
    t = pre * jax.nn.sigmoid(pre)
    z = jnp.dot(t.astype(jnp.bfloat16), w2_ref[...].astype(jnp.bfloat16),
                preferred_element_type=_F32) + b2_ref[...]
    ef = z * jax.nn.sigmoid(z)
    att = jax.nn.sigmoid(
        jnp.sum(ef * watt_ref[...], axis=-1, keepdims=True) + batt_ref[...])
    ef = ef * att
    tr = jnp.sum(ef * wp2_ref[...], axis=-1, keepdims=True)
    ef_ref[...] = ef
    one3 = (lax.broadcasted_iota(jnp.int32, dvec.shape, 1) == 3).astype(_F32)
    tr_ref[...] = dvec * tr + one3


def _edge_mlp(sm, dq, ea, w1e, wdsq, b1, w2, b2, watt, batt, wp2):
    blk = 8000
    full = lambda r, c: pl.BlockSpec((r, c), lambda i: (0, 0))
    return pl.pallas_call(
        _edge_body,
        grid=(E // blk,),
        in_specs=[
            pl.BlockSpec((blk, D), lambda i: (i, 0)),
            pl.BlockSpec((blk, 16), lambda i: (i, 0)),
            pl.BlockSpec((blk, DE), lambda i: (i, 0)),
            full(DE, D), full(1, D), full(1, D), full(D, D), full(1, D),
            full(1, D), full(1, 1), full(1, D),
        ],
        out_specs=[
            pl.BlockSpec((blk, D), lambda i: (i, 0)),
            pl.BlockSpec((blk, 16), lambda i: (i, 0)),
        ],
        out_shape=[
            jax.ShapeDtypeStruct((E, D), _F32),
            jax.ShapeDtypeStruct((E, 16), _F32),
        ],
    )(sm, dq, ea, w1e, wdsq, b1, w2, b2, watt, batt, wp2)


def _scatter_body(ef_hbm, tr_hbm, row_hbm, z1_hbm, z2_hbm,
                  p1_hbm, p2_hbm,
                  idx, bufs_e, bufs_t, acc1, acc2, rsems, ssems, isem):
    c = lax.axis_index("c")
    s = lax.axis_index("s")
    w = s * 2 + c
    pltpu.sync_copy(z1_hbm.at[pl.ds(s * NPT, NPT)], acc1.at[pl.ds(s * NPT, NPT)])
    pltpu.sync_copy(z2_hbm.at[pl.ds(s * NPT, NPT)], acc2.at[pl.ds(s * NPT, NPT)])

    @pl.loop(0, NSTEP)
    def _fill(j):
        pltpu.async_copy(row_hbm.at[pl.ds(w * EPW + j * CH, CH)], idx.at[j], isem)

    @pl.loop(0, NSTEP)
    def _drain(j):
        pltpu.make_async_copy(row_hbm.at[pl.ds(0, CH)], idx.at[0], isem).wait()

    plsc.subcore_barrier()

    def issue_read(t, base):
        pltpu.async_copy(ef_hbm.at[pl.ds(base, CH)], bufs_e[t], rsems.at[t])
        pltpu.async_copy(tr_hbm.at[pl.ds(base, CH)], bufs_t[t], rsems.at[t])

    def wait_read(t, base):
        pltpu.make_async_copy(ef_hbm.at[pl.ds(base, CH)], bufs_e[t], rsems.at[t]).wait()
        pltpu.make_async_copy(tr_hbm.at[pl.ds(base, CH)], bufs_t[t], rsems.at[t]).wait()

    def issue_scatter(t, j):
        pltpu.async_copy(bufs_e[t], acc1.at[idx.at[j]], ssems.at[t], add=True)
        pltpu.async_copy(bufs_t[t], acc2.at[idx.at[j]], ssems.at[t], add=True)

    def wait_scatter(t, j):
        pltpu.make_async_copy(bufs_e[t], acc1.at[idx.at[j]], ssems.at[t]).wait()
        pltpu.make_async_copy(bufs_t[t], acc2.at[idx.at[j]], ssems.at[t]).wait()

    issue_read(0, w * EPW)
    issue_read(1, w * EPW + CH)

    @pl.loop(0, (NSTEP - 1) // 2)
    def _pair(i):
        for t in range(2):
            jj = 2 * i + t
            base = w * EPW + jj * CH
            wait_read(t, base)
            issue_scatter(t, jj)
            wait_scatter(t, jj)

            @pl.when(jj + 2 < NSTEP)
            def _():
                issue_read(t, base + 2 * CH)

    jj = NSTEP - 1
    base = w * EPW + jj * CH
    wait_read(0, base)
    issue_scatter(0, jj)
    wait_scatter(0, jj)

    plsc.subcore_barrier()
    pltpu.sync_copy(acc1.at[pl.ds(s * NPT, NPT)], p1_hbm.at[c, pl.ds(s * NPT, NPT)])
    pltpu.sync_copy(acc2.at[pl.ds(s * NPT, NPT)], p2_hbm.at[c, pl.ds(s * NPT, NPT)])


def _scatter_sc(ef, tr, row1d, z1, z2):
    f = pl.kernel(
        _scatter_body,
        out_type=(
            jax.ShapeDtypeStruct((2, NP, D), _F32),
            jax.ShapeDtypeStruct((2, NP, 16), _F32),
        ),
        mesh=_MESH,
        scratch_types=[
            pltpu.VMEM((NSTEP, CH), jnp.int32),
            [pltpu.VMEM((CH, D), _F32)] * 2,
            [pltpu.VMEM((CH, 16), _F32)] * 2,
            pltpu.VMEM_SHARED((NP, D), _F32),
            pltpu.VMEM_SHARED((NP, 16), _F32),
            pltpu.SemaphoreType.DMA((2,)),
            pltpu.SemaphoreType.DMA((2,)),
            pltpu.SemaphoreType.DMA,
        ],
        compiler_params=_SC_PARAMS,
    )
    return f(ef, tr, row1d, z1, z2)


def _node_body(h_ref, pos_ref, a1_ref, a2_ref, t1_ref, t2_ref,
               wn1a_ref, wn1b_ref, bn1_ref, wn2_ref, bn2_ref,
               hn_ref, pn_ref):
    h = h_ref[...]
    agg = a1_ref[0] + a2_ref[0]
    t4 = t1_ref[0] + t2_ref[0]
    cnt = jnp.clip(t4[:, 3:4], 1.0, None)
    pn_ref[...] = pos_ref[...] + t4 / cnt
    pre = (jnp.dot(h, wn1a_ref[...], preferred_element_type=_F32)
           + jnp.dot(agg, wn1b_ref[...], preferred_element_type=_F32)
           + bn1_ref[...])
    nout = pre * jax.nn.sigmoid(pre)
    hn_ref[...] = (jnp.dot(nout, wn2_ref[...], preferred_element_type=_F32)
                   + bn2_ref[...] + h)


def _node_mlp(h, pos16, p1, p2, wn1a, wn1b, bn1, wn2, bn2):
    blk = 2000
    full = lambda r, c: pl.BlockSpec((r, c), lambda i: (0, 0))
    return pl.pallas_call(
        _node_body,
        grid=(N // blk,),
        in_specs=[
            pl.BlockSpec((blk, D), lambda i: (i, 0)),
            pl.BlockSpec((blk, 16), lambda i: (i, 0)),
            pl.BlockSpec((1, blk, D), lambda i: (0, i, 0)),
            pl.BlockSpec((1, blk, D), lambda i: (1, i, 0)),
            pl.BlockSpec((1, blk, 16), lambda i: (0, i, 0)),
            pl.BlockSpec((1, blk, 16), lambda i: (1, i, 0)),
            full(D, D), full(D, D), full(1, D), full(D, D), full(1, D),
        ],
        out_specs=[
            pl.BlockSpec((blk, D), lambda i: (i, 0)),
            pl.BlockSpec((blk, 16), lambda i: (i, 0)),
        ],
        out_shape=[
            jax.ShapeDtypeStruct((N, D), _F32),
            jax.ShapeDtypeStruct((N, 16), _F32),
        ],
    )(h, pos16, p1, p1, p2, p2, wn1a, wn1b, bn1, wn2, bn2)


def kernel(h, pos, edge_index, edge_attr,
           W_e1, b_e1, W_e2, b_e2, W_att, b_att,
           W_n1, b_n1, W_n2, b_n2, W_p1, b_p1, W_p2):
    ei = edge_index.astype(jnp.int32)
    row1d = ei[0]
    col1d = ei[1]
    pos16 = jnp.pad(pos, ((0, 0), (0, 13)))

    perm = jnp.array([32 * c + (j // 2 if j % 2 == 0 else 16 + j // 2)
                      for c in range(4) for j in range(32)], jnp.int32)
    w1a = W_e1[:D][:, perm]
    w1b = W_e1[D:2 * D][:, perm]
    wdsq = W_e1[2 * D:2 * D + 1]
    w1e = W_e1[2 * D + 1:]

    g1, g2 = _build_tables(h, w1a, w1b)
    sm, dq = _gather_sc(g1, g2, pos16, row1d, col1d)
    ef, tr = _edge_mlp(
        sm, dq, edge_attr,
        w1e, wdsq, b_e1.reshape(1, D), W_e2, b_e2.reshape(1, D),
        W_att.T, b_att.reshape(1, 1), W_p2.T)
    z1 = jnp.zeros((NP, D), _F32)
    z2 = jnp.zeros((NP, 16), _F32)
    p1, p2 = _scatter_sc(ef, tr, row1d, z1, z2)
    hn, pn16 = _node_mlp(
        h, pos16, p1, p2,
        W_n1[:D], W_n1[D:], b_n1.reshape(1, D), W_n2, b_n2.reshape(1, D))
    return hn, pn16[:, :3], ef


# speedup vs baseline: 1.1022x; 1.1022x over previous
import jax
import jax.numpy as jnp
from jax import lax
from jax.experimental import pallas as pl
from jax.experimental.pallas import tpu as pltpu
from jax.experimental.pallas import tpu_sc as plsc

N = 10000
E = 320000
D = 128
DE = 16
NW = 32
CH = 80
EPW = E // NW
NSTEP = EPW // CH
NP = 10240
NPT = NP // 16
EQ = E * 16 // 128
CHQ = CH * 16 // 128

_MESH = plsc.VectorSubcoreMesh(core_axis_name="c", subcore_axis_name="s")
_SC_PARAMS = pltpu.CompilerParams(use_tc_tiling_on_sc=False)
_F32 = jnp.float32


def _tables_body(h_ref, w1a_ref, w1b_ref, g1_ref, g2_ref):
    h = h_ref[...]
    g1_ref[...] = jnp.dot(h, w1a_ref[...], preferred_element_type=_F32)
    g2_ref[...] = jnp.dot(h, w1b_ref[...], preferred_element_type=_F32)


def _build_tables(h, w1a, w1b):
    blk = 2000
    return pl.pallas_call(
        _tables_body,
        grid=(N // blk,),
        in_specs=[
            pl.BlockSpec((blk, D), lambda i: (i, 0)),
            pl.BlockSpec((D, D), lambda i: (0, 0)),
            pl.BlockSpec((D, D), lambda i: (0, 0)),
        ],
        out_specs=[
            pl.BlockSpec((blk, D), lambda i: (i, 0)),
            pl.BlockSpec((blk, D), lambda i: (i, 0)),
        ],
        out_shape=[jax.ShapeDtypeStruct((N, D), _F32)] * 2,
    )(h, w1a, w1b)


def _gather_body(g1_hbm, g2_hbm, p_hbm, row_hbm, col_hbm,
                 s_hbm, dq_hbm,
                 idxr, idxc, bufs_a, bufs_b, bufs_c, bufs_d, bufs_q,
                 gsems, wsems):
    c = lax.axis_index("c")
    s = lax.axis_index("s")
    w = s * 2 + c
    pltpu.sync_copy(row_hbm.at[pl.ds(w * EPW, EPW)], idxr)
    pltpu.sync_copy(col_hbm.at[pl.ds(w * EPW, EPW)], idxc)

    def issue_gather(t, j):
        ir = idxr.at[pl.ds(j * CH, CH)]
        ic = idxc.at[pl.ds(j * CH, CH)]
        pltpu.async_copy(g1_hbm.at[ir], bufs_a[t], gsems.at[t])
        pltpu.async_copy(g2_hbm.at[ic], bufs_b[t], gsems.at[t])
        pltpu.async_copy(p_hbm.at[ir], bufs_c[t], gsems.at[t])
        pltpu.async_copy(p_hbm.at[ic], bufs_d[t], gsems.at[t])

    def wait_gather(t, j):
        ir = idxr.at[pl.ds(j * CH, CH)]
        ic = idxc.at[pl.ds(j * CH, CH)]
        pltpu.make_async_copy(g1_hbm.at[ir], bufs_a[t], gsems.at[t]).wait()
        pltpu.make_async_copy(g2_hbm.at[ic], bufs_b[t], gsems.at[t]).wait()
        pltpu.make_async_copy(p_hbm.at[ir], bufs_c[t], gsems.at[t]).wait()
        pltpu.make_async_copy(p_hbm.at[ic], bufs_d[t], gsems.at[t]).wait()

    def combine(t):
        a, b, pc, pd, q = bufs_a[t], bufs_b[t], bufs_c[t], bufs_d[t], bufs_q[t]

        @pl.loop(0, CHQ)
        def _rows(m):
            for k in range(8):
                i = m * 8 + k
                for cc in range(8):
                    sl = pl.ds(cc * 16, 16)
                    a[i, sl] = a[i, sl] + b[i, sl]
                q[i, :] = pc[i, :] - pd[i, :]

    def issue_writes(t, base):
        pltpu.async_copy(bufs_a[t], s_hbm.at[pl.ds(base, CH)], wsems.at[t])
        pltpu.async_copy(bufs_q[t], dq_hbm.at[pl.ds(base, CH)], wsems.at[t])

    def wait_writes(t, base):
        pltpu.make_async_copy(bufs_a[t], s_hbm.at[pl.ds(base, CH)], wsems.at[t]).wait()
        pltpu.make_async_copy(bufs_q[t], dq_hbm.at[pl.ds(base, CH)], wsems.at[t]).wait()

    issue_gather(0, 0)
    issue_gather(1, 1)

    @pl.loop(0, (NSTEP - 1) // 2)
    def _pair(i):
        for t in range(2):
            jj = 2 * i + t
            base = w * EPW + jj * CH
            wait_gather(t, jj)
            combine(t)
            issue_writes(t, base)
            wait_writes(t, base)

            @pl.when(jj + 2 < NSTEP)
            def _():
                issue_gather(t, jj + 2)

    jj = NSTEP - 1
    base = w * EPW + jj * CH
    wait_gather(0, jj)
    combine(0)
    issue_writes(0, base)
    wait_writes(0, base)


def _gather_sc(g1, g2, pos16, row1d, col1d):
    f = pl.kernel(
        _gather_body,
        out_type=(
            jax.ShapeDtypeStruct((E, D), _F32),
            jax.ShapeDtypeStruct((E, 16), _F32),
        ),
        mesh=_MESH,
        scratch_types=[
            pltpu.VMEM((EPW,), jnp.int32),
            pltpu.VMEM((EPW,), jnp.int32),
            [pltpu.VMEM((CH, D), _F32)] * 2,
            [pltpu.VMEM((CH, D), _F32)] * 2,
            [pltpu.VMEM((CH, 16), _F32)] * 2,
            [pltpu.VMEM((CH, 16), _F32)] * 2,
            [pltpu.VMEM((CH, 16), _F32)] * 2,
            pltpu.SemaphoreType.DMA((2,)),
            pltpu.SemaphoreType.DMA((2,)),
        ],
        compiler_params=_SC_PARAMS,
    )
    return f(g1, g2, pos16, row1d, col1d)


def _edge_body(s_ref, dq_ref, ea_ref,
               w1e_ref, wdsq_ref, b1_ref, w2_ref, b2_ref,
               watt_ref, batt_ref, wp2_ref,
               ef_ref, tr_ref):
    dvec = dq_ref[...]
    dsq = jnp.sum(dvec * dvec, axis=-1, keepdims=True)
    pre = (s_ref[...] + dsq * wdsq_ref[...] + b1_ref[...]
           + jnp.dot(ea_ref[...], w1e_ref[...], preferred_element_type=_F32))
    t = pre * jax.nn.sigmoid(pre)
    z = jnp.dot(t.astype(jnp.bfloat16), w2_ref[...].astype(jnp.bfloat16),
                preferred_element_type=_F32) + b2_ref[...]
    ef = z * jax.nn.sigmoid(z)
    att = jax.nn.sigmoid(
        jnp.sum(ef * watt_ref[...], axis=-1, keepdims=True) + batt_ref[...])
    ef = ef * att
    tr = jnp.sum(ef * wp2_ref[...], axis=-1, keepdims=True)
    ef_ref[...] = ef
    one3 = (lax.broadcasted_iota(jnp.int32, dvec.shape, 1) == 3).astype(_F32)
    tr_ref[...] = dvec * tr + one3


def _edge_mlp(sm, dq, ea, w1e, wdsq, b1, w2, b2, watt, batt, wp2):
    blk = 8000
    full = lambda r, c: pl.BlockSpec((r, c), lambda i: (0, 0))
    return pl.pallas_call(
        _edge_body,
        grid=(E // blk,),
        in_specs=[
            pl.BlockSpec((blk, D), lambda i: (i, 0)),
            pl.BlockSpec((blk, 16), lambda i: (i, 0)),
            pl.BlockSpec((blk, DE), lambda i: (i, 0)),
            full(DE, D), full(1, D), full(1, D), full(D, D), full(1, D),
            full(1, D), full(1, 1), full(1, D),
        ],
        out_specs=[
            pl.BlockSpec((blk, D), lambda i: (i, 0)),
            pl.BlockSpec((blk, 16), lambda i: (i, 0)),
        ],
        out_shape=[
            jax.ShapeDtypeStruct((E, D), _F32),
            jax.ShapeDtypeStruct((E, 16), _F32),
        ],
    )(sm, dq, ea, w1e, wdsq, b1, w2, b2, watt, batt, wp2)


def _scatter_body(ef_hbm, tr_hbm, row_hbm, z1_hbm, z2_hbm,
                  p1_hbm, p2_hbm,
                  idx, bufs_e, bufs_t, acc1, acc2, rsems, ssems, isem):
    c = lax.axis_index("c")
    s = lax.axis_index("s")
    w = s * 2 + c
    pltpu.sync_copy(z1_hbm.at[pl.ds(s * NPT, NPT)], acc1.at[pl.ds(s * NPT, NPT)])
    pltpu.sync_copy(z2_hbm.at[pl.ds(s * NPT, NPT)], acc2.at[pl.ds(s * NPT, NPT)])

    @pl.loop(0, NSTEP)
    def _fill(j):
        pltpu.async_copy(row_hbm.at[pl.ds(w * EPW + j * CH, CH)], idx.at[j], isem)

    @pl.loop(0, NSTEP)
    def _drain(j):
        pltpu.make_async_copy(row_hbm.at[pl.ds(0, CH)], idx.at[0], isem).wait()

    plsc.subcore_barrier()

    def issue_read(t, base):
        pltpu.async_copy(ef_hbm.at[pl.ds(base, CH)], bufs_e[t], rsems.at[t])
        pltpu.async_copy(tr_hbm.at[pl.ds(base, CH)], bufs_t[t], rsems.at[t])

    def wait_read(t, base):
        pltpu.make_async_copy(ef_hbm.at[pl.ds(base, CH)], bufs_e[t], rsems.at[t]).wait()
        pltpu.make_async_copy(tr_hbm.at[pl.ds(base, CH)], bufs_t[t], rsems.at[t]).wait()

    def issue_scatter(t, j):
        pltpu.async_copy(bufs_e[t], acc1.at[idx.at[j]], ssems.at[t], add=True)
        pltpu.async_copy(bufs_t[t], acc2.at[idx.at[j]], ssems.at[t], add=True)

    def wait_scatter(t, j):
        pltpu.make_async_copy(bufs_e[t], acc1.at[idx.at[j]], ssems.at[t]).wait()
        pltpu.make_async_copy(bufs_t[t], acc2.at[idx.at[j]], ssems.at[t]).wait()

    issue_read(0, w * EPW)
    issue_read(1, w * EPW + CH)

    @pl.loop(0, (NSTEP - 1) // 2)
    def _pair(i):
        for t in range(2):
            jj = 2 * i + t
            base = w * EPW + jj * CH
            wait_read(t, base)
            issue_scatter(t, jj)
            wait_scatter(t, jj)

            @pl.when(jj + 2 < NSTEP)
            def _():
                issue_read(t, base + 2 * CH)

    jj = NSTEP - 1
    base = w * EPW + jj * CH
    wait_read(0, base)
    issue_scatter(0, jj)
    wait_scatter(0, jj)

    plsc.subcore_barrier()
    pltpu.sync_copy(acc1.at[pl.ds(s * NPT, NPT)], p1_hbm.at[c, pl.ds(s * NPT, NPT)])
    pltpu.sync_copy(acc2.at[pl.ds(s * NPT, NPT)], p2_hbm.at[c, pl.ds(s * NPT, NPT)])


def _scatter_sc(ef, tr, row1d, z1, z2):
    f = pl.kernel(
        _scatter_body,
        out_type=(
            jax.ShapeDtypeStruct((2, NP, D), _F32),
            jax.ShapeDtypeStruct((2, NP, 16), _F32),
        ),
        mesh=_MESH,
        scratch_types=[
            pltpu.VMEM((NSTEP, CH), jnp.int32),
            [pltpu.VMEM((CH, D), _F32)] * 2,
            [pltpu.VMEM((CH, 16), _F32)] * 2,
            pltpu.VMEM_SHARED((NP, D), _F32),
            pltpu.VMEM_SHARED((NP, 16), _F32),
            pltpu.SemaphoreType.DMA((2,)),
            pltpu.SemaphoreType.DMA((2,)),
            pltpu.SemaphoreType.DMA,
        ],
        compiler_params=_SC_PARAMS,
    )
    return f(ef, tr, row1d, z1, z2)


def _node_body(h_ref, pos_ref, a1_ref, a2_ref, t1_ref, t2_ref,
               wn1a_ref, wn1b_ref, bn1_ref, wn2_ref, bn2_ref,
               hn_ref, pn_ref):
    h = h_ref[...]
    agg = a1_ref[0] + a2_ref[0]
    t4 = t1_ref[0] + t2_ref[0]
    cnt = jnp.clip(t4[:, 3:4], 1.0, None)
    pn_ref[...] = pos_ref[...] + t4 / cnt
    pre = (jnp.dot(h, wn1a_ref[...], preferred_element_type=_F32)
           + jnp.dot(agg, wn1b_ref[...], preferred_element_type=_F32)
           + bn1_ref[...])
    nout = pre * jax.nn.sigmoid(pre)
    hn_ref[...] = (jnp.dot(nout, wn2_ref[...], preferred_element_type=_F32)
                   + bn2_ref[...] + h)


def _node_mlp(h, pos16, p1, p2, wn1a, wn1b, bn1, wn2, bn2):
    blk = 2000
    full = lambda r, c: pl.BlockSpec((r, c), lambda i: (0, 0))
    return pl.pallas_call(
        _node_body,
        grid=(N // blk,),
        in_specs=[
            pl.BlockSpec((blk, D), lambda i: (i, 0)),
            pl.BlockSpec((blk, 16), lambda i: (i, 0)),
            pl.BlockSpec((1, blk, D), lambda i: (0, i, 0)),
            pl.BlockSpec((1, blk, D), lambda i: (1, i, 0)),
            pl.BlockSpec((1, blk, 16), lambda i: (0, i, 0)),
            pl.BlockSpec((1, blk, 16), lambda i: (1, i, 0)),
            full(D, D), full(D, D), full(1, D), full(D, D), full(1, D),
        ],
        out_specs=[
            pl.BlockSpec((blk, D), lambda i: (i, 0)),
            pl.BlockSpec((blk, 16), lambda i: (i, 0)),
        ],
        out_shape=[
            jax.ShapeDtypeStruct((N, D), _F32),
            jax.ShapeDtypeStruct((N, 16), _F32),
        ],
    )(h, pos16, p1, p1, p2, p2, wn1a, wn1b, bn1, wn2, bn2)


def kernel(h, pos, edge_index, edge_attr,
           W_e1, b_e1, W_e2, b_e2, W_att, b_att,
           W_n1, b_n1, W_n2, b_n2, W_p1, b_p1, W_p2):
    ei = edge_index.astype(jnp.int32)
    row1d = ei[0]
    col1d = ei[1]
    pos16 = jnp.pad(pos, ((0, 0), (0, 13)))

    w1a = W_e1[:D]
    w1b = W_e1[D:2 * D]
    wdsq = W_e1[2 * D:2 * D + 1]
    w1e = W_e1[2 * D + 1:]

    g1, g2 = _build_tables(h, w1a, w1b)
    sm, dq = _gather_sc(g1, g2, pos16, row1d, col1d)
    ef, tr = _edge_mlp(
        sm, dq, edge_attr,
        w1e, wdsq, b_e1.reshape(1, D), W_e2, b_e2.reshape(1, D),
        W_att.T, b_att.reshape(1, 1), W_p2.T)
    z1 = jnp.zeros((NP, D), _F32)
    z2 = jnp.zeros((NP, 16), _F32)
    p1, p2 = _scatter_sc(ef, tr, row1d, z1, z2)
    hn, pn16 = _node_mlp(
        h, pos16, p1, p2,
        W_n1[:D], W_n1[D:], b_n1.reshape(1, D), W_n2, b_n2.reshape(1, D))
    return hn, pn16[:, :3], ef
